# baseline (speedup 1.0000x reference)
import functools

import jax
import jax.numpy as jnp
from jax.experimental import pallas as pl
from jax.experimental.pallas import tpu as pltpu
from jax.experimental.pallas import tpu_sc as plsc

_B, _N, _M, _C = 4, 8192, 2048, 64
_TN = 512

_NSC = 32
_QPW = (_B * _N) // _NSC
_QCH = 256
_NCH = _QPW // _QCH
_RPC = 3 * _QCH


def _knn_body(u_ref, k_ref, idx_ref, w_ref):
    u = u_ref[0]
    kp = k_ref[0]

    d2 = jnp.zeros((_TN, _M), jnp.float32)
    for d in range(3):
        diff = u[:, d][:, None] - kp[d, :][None, :]
        d2 = d2 + diff * diff

    iota = jax.lax.broadcasted_iota(jnp.int32, (_TN, _M), 1)
    dcur = d2
    vals, idxs = [], []
    for k in range(3):
        mn = jnp.min(dcur, axis=1, keepdims=True)
        am = jnp.min(jnp.where(dcur == mn, iota, _M), axis=1, keepdims=True)
        vals.append(mn)
        idxs.append(am)
        if k < 2:
            dcur = jnp.where(iota == am, jnp.float32(jnp.inf), dcur)

    recips = [1.0 / (jnp.sqrt(jnp.maximum(v, 0.0)) + 1e-8) for v in vals]
    norm = (recips[0] + recips[1]) + recips[2]

    b = pl.program_id(0)
    idx_ref[0] = jnp.concatenate(idxs, axis=1) + b * _M
    w_ref[0] = jnp.concatenate([r / norm for r in recips], axis=1)


def _interp_body(idx_hbm, w_hbm, f_hbm, out_hbm, idx_v, w_v, rows_v,
                 out_scr, sem):
    wid = jax.lax.axis_index("s") * 2 + jax.lax.axis_index("c")
    b = wid // 8
    qbase = (wid % 8) * _QPW

    lanes = jax.lax.iota(jnp.int32, 16)

    for ci in range(_NCH):
        q0 = qbase + ci * _QCH
        pltpu.sync_copy(idx_hbm.at[b, pl.ds(q0 * 3, _RPC)], idx_v)
        pltpu.sync_copy(w_hbm.at[b, pl.ds(q0 * 3, _RPC)], w_v)
        cps = [pltpu.async_copy(f_hbm.at[idx_v.at[pl.ds(j * 128, 128)]],
                                rows_v.at[pl.ds(j * 128, 128)], sem)
               for j in range(_RPC // 128)]
        for cp in cps:
            cp.wait()

        def group(g, carry):
            lq = g * 16 + lanes
            iq0 = lq * 3
            iq1 = iq0 + 1
            iq2 = iq0 + 2
            w0 = plsc.load_gather(w_v, [iq0])
            w1 = plsc.load_gather(w_v, [iq1])
            w2 = plsc.load_gather(w_v, [iq2])
            for c in range(_C):
                cv = jnp.full((16,), c, jnp.int32)
                f0 = plsc.load_gather(rows_v, [iq0, cv])
                f1 = plsc.load_gather(rows_v, [iq1, cv])
                f2 = plsc.load_gather(rows_v, [iq2, cv])
                out_scr[c, pl.ds(g * 16, 16)] = (w0 * f0 + w1 * f1) + w2 * f2
            return carry

        jax.lax.fori_loop(0, _QCH // 16, group, 0)
        pltpu.sync_copy(out_scr, out_hbm.at[b, :, pl.ds(q0, _QCH)])


def kernel(unknown, known, known_feats):
    known_t = jnp.transpose(known, (0, 2, 1))
    idx, wgt = pl.pallas_call(
        _knn_body,
        grid=(_B, _N // _TN),
        in_specs=[
            pl.BlockSpec((1, _TN, 3), lambda b, i: (b, i, 0)),
            pl.BlockSpec((1, 3, _M), lambda b, i: (b, 0, 0)),
        ],
        out_specs=[
            pl.BlockSpec((1, _TN, 3), lambda b, i: (b, i, 0)),
            pl.BlockSpec((1, _TN, 3), lambda b, i: (b, i, 0)),
        ],
        out_shape=[
            jax.ShapeDtypeStruct((_B, _N, 3), jnp.int32),
            jax.ShapeDtypeStruct((_B, _N, 3), jnp.float32),
        ],
    )(unknown, known_t)

    idxf = idx.reshape(_B, _N * 3)
    wflat = wgt.reshape(_B, _N * 3)
    feats_flat = jnp.transpose(known_feats, (0, 2, 1)).reshape(_B * _M, _C)

    mesh = plsc.VectorSubcoreMesh(core_axis_name="c", subcore_axis_name="s")
    interp = functools.partial(
        pl.kernel,
        mesh=mesh,
        compiler_params=pltpu.CompilerParams(needs_layout_passes=False,
                                             use_tc_tiling_on_sc=False),
        out_type=jax.ShapeDtypeStruct((_B, _C, _N), jnp.float32),
        scratch_types=[
            pltpu.VMEM((_RPC,), jnp.int32),
            pltpu.VMEM((_RPC,), jnp.float32),
            pltpu.VMEM((_RPC, _C), jnp.float32),
            pltpu.VMEM((_C, _QCH), jnp.float32),
            pltpu.SemaphoreType.DMA,
        ],
    )(_interp_body)
    return interp(idxf, wflat, feats_flat)


# speedup vs baseline: 1.2186x; 1.2186x over previous
import functools

import jax
import jax.numpy as jnp
from jax.experimental import pallas as pl
from jax.experimental.pallas import tpu as pltpu
from jax.experimental.pallas import tpu_sc as plsc

_B, _N, _M, _C = 4, 8192, 2048, 64
_TN = 512

_NSC = 32
_QCH = _N // _NSC
_RPC = 3 * _QCH


def _knn_body(u_ref, k_ref, idx_ref, w_ref):
    u = u_ref[...]
    kp = k_ref[...]

    d2 = jnp.zeros((_TN, _M), jnp.float32)
    for d in range(3):
        diff = u[:, d][:, None] - kp[d, :][None, :]
        d2 = d2 + diff * diff

    iota = jax.lax.broadcasted_iota(jnp.int32, (_TN, _M), 1)
    dcur = d2
    vals, idxs = [], []
    for k in range(3):
        mn = jnp.min(dcur, axis=1, keepdims=True)
        am = jnp.min(jnp.where(dcur == mn, iota, _M), axis=1, keepdims=True)
        vals.append(mn)
        idxs.append(am)
        if k < 2:
            dcur = jnp.where(iota == am, jnp.float32(jnp.inf), dcur)

    recips = [1.0 / (jnp.sqrt(jnp.maximum(v, 0.0)) + 1e-8) for v in vals]
    norm = (recips[0] + recips[1]) + recips[2]

    idx_ref[...] = jnp.concatenate(idxs, axis=1)
    w_ref[...] = jnp.concatenate([r / norm for r in recips], axis=1)


def _interp_body(idx_hbm, w_hbm, f_hbm, out_hbm, idx_v, w_v, rows_v,
                 out_scr, sem):
    wid = jax.lax.axis_index("s") * 2 + jax.lax.axis_index("c")
    q0 = wid * _QCH

    lanes = jax.lax.iota(jnp.int32, 16)

    pltpu.sync_copy(idx_hbm.at[pl.ds(q0 * 3, _RPC)], idx_v)
    pltpu.sync_copy(w_hbm.at[pl.ds(q0 * 3, _RPC)], w_v)
    cps = [pltpu.async_copy(f_hbm.at[idx_v.at[pl.ds(j * 128, 128)]],
                            rows_v.at[pl.ds(j * 128, 128)], sem)
           for j in range(_RPC // 128)]
    for cp in cps:
        cp.wait()

    def group(g, carry):
        lq = g * 16 + lanes
        iq0 = lq * 3
        iq1 = iq0 + 1
        iq2 = iq0 + 2
        w0 = plsc.load_gather(w_v, [iq0])
        w1 = plsc.load_gather(w_v, [iq1])
        w2 = plsc.load_gather(w_v, [iq2])
        for c in range(_C):
            cv = jnp.full((16,), c, jnp.int32)
            f0 = plsc.load_gather(rows_v, [iq0, cv])
            f1 = plsc.load_gather(rows_v, [iq1, cv])
            f2 = plsc.load_gather(rows_v, [iq2, cv])
            out_scr[c, pl.ds(g * 16, 16)] = (w0 * f0 + w1 * f1) + w2 * f2
        return carry

    jax.lax.fori_loop(0, _QCH // 16, group, 0)
    pltpu.sync_copy(out_scr, out_hbm.at[:, pl.ds(q0, _QCH)])


def kernel(unknown, known, known_feats):
    known_t = jnp.transpose(known, (0, 2, 1))
    feats_t = jnp.transpose(known_feats, (0, 2, 1))

    knn = pl.pallas_call(
        _knn_body,
        grid=(_N // _TN,),
        in_specs=[
            pl.BlockSpec((_TN, 3), lambda i: (i, 0)),
            pl.BlockSpec((3, _M), lambda i: (0, 0)),
        ],
        out_specs=[
            pl.BlockSpec((_TN, 3), lambda i: (i, 0)),
            pl.BlockSpec((_TN, 3), lambda i: (i, 0)),
        ],
        out_shape=[
            jax.ShapeDtypeStruct((_N, 3), jnp.int32),
            jax.ShapeDtypeStruct((_N, 3), jnp.float32),
        ],
    )

    mesh = plsc.VectorSubcoreMesh(core_axis_name="c", subcore_axis_name="s")
    interp = functools.partial(
        pl.kernel,
        mesh=mesh,
        compiler_params=pltpu.CompilerParams(needs_layout_passes=False,
                                             use_tc_tiling_on_sc=False),
        out_type=jax.ShapeDtypeStruct((_C, _N), jnp.float32),
        scratch_types=[
            pltpu.VMEM((_RPC,), jnp.int32),
            pltpu.VMEM((_RPC,), jnp.float32),
            pltpu.VMEM((_RPC, _C), jnp.float32),
            pltpu.VMEM((_C, _QCH), jnp.float32),
            pltpu.SemaphoreType.DMA,
        ],
    )(_interp_body)

    outs = []
    for b in range(_B):
        idx, wgt = knn(unknown[b], known_t[b])
        outs.append(interp(idx.reshape(_N * 3), wgt.reshape(_N * 3),
                           feats_t[b]))
    return jnp.stack(outs)


# speedup vs baseline: 1.2969x; 1.0643x over previous
import functools

import jax
import jax.numpy as jnp
from jax.experimental import pallas as pl
from jax.experimental.pallas import tpu as pltpu
from jax.experimental.pallas import tpu_sc as plsc

_B, _N, _M, _C = 4, 8192, 2048, 64
_TN = 512

_NSC = 32
_QCH = _N // _NSC
_RPC = 3 * _QCH


def _knn_body(u_ref, k_ref, idx_ref, w_ref):
    u = u_ref[...]
    kp = k_ref[...]

    d2 = jnp.zeros((_TN, _M), jnp.float32)
    for d in range(3):
        diff = u[:, d][:, None] - kp[d, :][None, :]
        d2 = d2 + diff * diff

    iota = jax.lax.broadcasted_iota(jnp.int32, (_TN, _M), 1)
    dcur = d2
    vals, idxs = [], []
    for k in range(3):
        mn = jnp.min(dcur, axis=1, keepdims=True)
        am = jnp.min(jnp.where(dcur == mn, iota, _M), axis=1, keepdims=True)
        vals.append(mn)
        idxs.append(am)
        if k < 2:
            dcur = jnp.where(iota == am, jnp.float32(jnp.inf), dcur)

    recips = [1.0 / (jnp.sqrt(jnp.maximum(v, 0.0)) + 1e-8) for v in vals]
    norm = (recips[0] + recips[1]) + recips[2]

    idx_ref[...] = jnp.concatenate(idxs, axis=1)
    w_ref[...] = jnp.concatenate([r / norm for r in recips], axis=1)


def _interp_body(idx_hbm, w_hbm, f_hbm, out_hbm, idx_v, w_v, rows_v,
                 out_scr, sem):
    wid = jax.lax.axis_index("s") * 2 + jax.lax.axis_index("c")
    q0 = wid * _QCH

    lanes = jax.lax.iota(jnp.int32, 16)

    pltpu.sync_copy(idx_hbm.at[pl.ds(q0 * 3, _RPC)], idx_v)
    pltpu.sync_copy(w_hbm.at[pl.ds(q0 * 3, _RPC)], w_v)
    cps = [pltpu.async_copy(f_hbm.at[idx_v.at[pl.ds(j * 128, 128)]],
                            rows_v.at[pl.ds(j * 128, 128)], sem)
           for j in range(_RPC // 128)]
    for cp in cps:
        cp.wait()

    rowv = [cb * 16 + lanes for cb in range(_C // 16)]

    def group(g, carry):
        for j in range(16):
            q = g * 16 + j
            i0 = 3 * q
            w0 = plsc.load_gather(w_v, [jnp.full((16,), i0, jnp.int32)])
            w1 = plsc.load_gather(w_v, [jnp.full((16,), i0 + 1, jnp.int32)])
            w2 = plsc.load_gather(w_v, [jnp.full((16,), i0 + 2, jnp.int32)])
            qv = jnp.full((16,), q, jnp.int32)
            for cb in range(_C // 16):
                sl = pl.ds(cb * 16, 16)
                acc = (w0 * rows_v[i0, sl] + w1 * rows_v[i0 + 1, sl]) \
                    + w2 * rows_v[i0 + 2, sl]
                plsc.store_scatter(out_scr, [rowv[cb], qv], acc)
        return carry

    jax.lax.fori_loop(0, _QCH // 16, group, 0)
    pltpu.sync_copy(out_scr.at[:, pl.ds(0, _QCH)],
                    out_hbm.at[:, pl.ds(q0, _QCH)])


def kernel(unknown, known, known_feats):
    known_t = jnp.transpose(known, (0, 2, 1))
    feats_t = jnp.transpose(known_feats, (0, 2, 1))

    knn = pl.pallas_call(
        _knn_body,
        grid=(_N // _TN,),
        in_specs=[
            pl.BlockSpec((_TN, 3), lambda i: (i, 0)),
            pl.BlockSpec((3, _M), lambda i: (0, 0)),
        ],
        out_specs=[
            pl.BlockSpec((_TN, 3), lambda i: (i, 0)),
            pl.BlockSpec((_TN, 3), lambda i: (i, 0)),
        ],
        out_shape=[
            jax.ShapeDtypeStruct((_N, 3), jnp.int32),
            jax.ShapeDtypeStruct((_N, 3), jnp.float32),
        ],
    )

    mesh = plsc.VectorSubcoreMesh(core_axis_name="c", subcore_axis_name="s")
    interp = functools.partial(
        pl.kernel,
        mesh=mesh,
        compiler_params=pltpu.CompilerParams(needs_layout_passes=False,
                                             use_tc_tiling_on_sc=False),
        out_type=jax.ShapeDtypeStruct((_C, _N), jnp.float32),
        scratch_types=[
            pltpu.VMEM((_RPC,), jnp.int32),
            pltpu.VMEM((_RPC,), jnp.float32),
            pltpu.VMEM((_RPC, _C), jnp.float32),
            pltpu.VMEM((_C, _QCH + 1), jnp.float32),
            pltpu.SemaphoreType.DMA,
        ],
    )(_interp_body)

    outs = []
    for b in range(_B):
        idx, wgt = knn(unknown[b], known_t[b])
        outs.append(interp(idx.reshape(_N * 3), wgt.reshape(_N * 3),
                           feats_t[b]))
    return jnp.stack(outs)
